# Initial kernel scaffold; baseline (speedup 1.0000x reference)
import jax
import jax.numpy as jnp
from jax.experimental import pallas as pl


def kernel(content_features, training_features, W1, b1, W2, b2, target_speaker_id, speaker_ids):
    raise NotImplementedError("write your pallas kernel here")


# speedup vs baseline: 1.6008x; 1.6008x over previous
import functools

import jax
import jax.numpy as jnp
from jax import lax
from jax.experimental import pallas as pl
from jax.experimental.pallas import tpu as pltpu
from jax.experimental.pallas import tpu_sc as plsc

B, D, K, TOPK = 256, 768, 100000, 5
KB = 2000
NBLK = K // KB
NCAND = KB + 128
EPS = 1e-8
NEG_INF = float("-inf")


def _topk_body(content_ref, keys_ref, spk_ref, tgt_ref, idx_out, w_out,
               run_vals, run_idx):
    pid = pl.program_id(0)

    @pl.when(pid == 0)
    def _init():
        run_vals[...] = jnp.full((B, 128), NEG_INF, jnp.float32)
        run_idx[...] = jnp.zeros((B, 128), jnp.int32)

    content = content_ref[...]
    rn_q = 1.0 / (jnp.sqrt(jnp.sum(content * content, axis=1, keepdims=True)) + EPS)
    qn = content * rn_q

    keys = keys_ref[...]
    rn_k = 1.0 / (jnp.sqrt(jnp.sum(keys * keys, axis=1, keepdims=True)) + EPS)
    kn = keys * rn_k

    sims = lax.dot_general(qn, kn, (((1,), (1,)), ((), ())),
                           preferred_element_type=jnp.float32)

    spk = spk_ref[0]
    tgt = tgt_ref[...]
    masked = jnp.where(spk == tgt, sims, NEG_INF)

    ri_old = run_idx[...]
    buf = jnp.concatenate([masked, run_vals[...]], axis=1)
    lane = lax.broadcasted_iota(jnp.int32, (B, NCAND), 1)
    lane128 = lax.broadcasted_iota(jnp.int32, (B, 128), 1)

    nv = jnp.full((B, 128), NEG_INF, jnp.float32)
    ni = jnp.zeros((B, 128), jnp.int32)
    for t in range(TOPK):
        v = jnp.max(buf, axis=1, keepdims=True)
        a = jnp.argmax(buf, axis=1).reshape(B, 1)
        run_sel = jnp.zeros((B, 1), jnp.int32)
        for j in range(TOPK):
            run_sel = jnp.where(a == KB + j, ri_old[:, j:j + 1], run_sel)
        gidx = jnp.where(a < KB, pid * KB + a, run_sel)
        nv = jnp.where(lane128 == t, v, nv)
        ni = jnp.where(lane128 == t, gidx, ni)
        buf = jnp.where(lane == a, NEG_INF, buf)

    run_vals[...] = nv
    run_idx[...] = ni

    @pl.when(pid == NBLK - 1)
    def _fini():
        finite = (nv > NEG_INF) & (lane128 < TOPK)
        valid = jnp.where(finite, 1.0, 0.0)
        denom = jnp.maximum(jnp.sum(valid, axis=1, keepdims=True), 1.0)
        w = valid / denom
        idx_out[...] = ni[:, :TOPK]
        w_out[...] = w[:, :TOPK]


def _masked_topk(content, keys, spk, tgt):
    spk3 = spk.reshape(NBLK, 1, KB)
    tgt2 = tgt.reshape(B, 1)
    return pl.pallas_call(
        _topk_body,
        grid=(NBLK,),
        in_specs=[
            pl.BlockSpec((B, D), lambda i: (0, 0)),
            pl.BlockSpec((KB, D), lambda i: (i, 0)),
            pl.BlockSpec((1, 1, KB), lambda i: (i, 0, 0)),
            pl.BlockSpec((B, 1), lambda i: (0, 0)),
        ],
        out_specs=[
            pl.BlockSpec((B, TOPK), lambda i: (0, 0)),
            pl.BlockSpec((B, TOPK), lambda i: (0, 0)),
        ],
        out_shape=[
            jax.ShapeDtypeStruct((B, TOPK), jnp.int32),
            jax.ShapeDtypeStruct((B, TOPK), jnp.float32),
        ],
        scratch_shapes=[
            pltpu.VMEM((B, 128), jnp.float32),
            pltpu.VMEM((B, 128), jnp.int32),
        ],
        compiler_params=pltpu.CompilerParams(
            dimension_semantics=("arbitrary",)),
    )(content, keys, spk3, tgt2)


def _sc_gather(table, idx_flat):
    info = plsc.get_sparse_core_info()
    nc, ns = info.num_cores, info.num_subcores
    nw = nc * ns
    n = idx_flat.shape[0]
    b_per_w = n // nw
    mesh = plsc.VectorSubcoreMesh(core_axis_name="c", subcore_axis_name="s")

    @functools.partial(
        pl.kernel,
        mesh=mesh,
        out_type=jax.ShapeDtypeStruct((n, D), jnp.float32),
        scratch_types=[
            pltpu.VMEM((b_per_w,), jnp.int32),
            pltpu.VMEM((b_per_w, D), jnp.float32),
            pltpu.SemaphoreType.DMA,
        ],
    )
    def gather_kernel(table_hbm, idx_hbm, out_hbm, idx_v, rows_v, sem):
        wid = lax.axis_index("s") * nc + lax.axis_index("c")
        base = wid * b_per_w
        pltpu.sync_copy(idx_hbm.at[pl.ds(base, b_per_w)], idx_v)
        pltpu.async_copy(table_hbm.at[idx_v], rows_v, sem).wait()
        pltpu.sync_copy(rows_v, out_hbm.at[pl.ds(base, b_per_w)])

    return gather_kernel(table, idx_flat)


def _mlp_body(content_ref, g_ref, w_ref, W1_ref, b1_ref, W2_ref, b2_ref,
              out_ref):
    content = content_ref[...]
    w = w_ref[...]
    rmean = jnp.zeros((B, D), jnp.float32)
    for t in range(TOPK):
        rmean = rmean + g_ref[t] * w[:, t:t + 1]
    has_any = jnp.sum(w, axis=1, keepdims=True) > 0.5

    W1 = W1_ref[...]
    h = lax.dot_general(content, W1[:, :D], (((1,), (1,)), ((), ())),
                        preferred_element_type=jnp.float32)
    h = h + lax.dot_general(rmean, W1[:, D:], (((1,), (1,)), ((), ())),
                            preferred_element_type=jnp.float32)
    h = jnp.maximum(h + b1_ref[...], 0.0)
    out = lax.dot_general(h, W2_ref[...], (((1,), (1,)), ((), ())),
                          preferred_element_type=jnp.float32)
    out = out + b2_ref[...]
    out_ref[...] = jnp.where(has_any, out, content)


def _mlp(content, gathered, w, W1, b1, W2, b2):
    return pl.pallas_call(
        _mlp_body,
        out_shape=jax.ShapeDtypeStruct((B, D), jnp.float32),
    )(content, gathered, w, W1, b1.reshape(1, D), W2, b2.reshape(1, D))


@jax.jit
def kernel(content_features, training_features, W1, b1, W2, b2,
           target_speaker_id, speaker_ids):
    top_idx, top_w = _masked_topk(content_features, training_features,
                                  speaker_ids.astype(jnp.int32),
                                  target_speaker_id.astype(jnp.int32))
    idx_flat = top_idx.T.reshape(B * TOPK)
    gathered = _sc_gather(training_features, idx_flat)
    g3 = gathered.reshape(TOPK, B, D)
    return _mlp(content_features, g3, top_w, W1, b1, W2, b2)


# speedup vs baseline: 2.7368x; 1.7097x over previous
import functools

import jax
import jax.numpy as jnp
from jax import lax
from jax.experimental import pallas as pl
from jax.experimental.pallas import tpu as pltpu
from jax.experimental.pallas import tpu_sc as plsc

B, D, K, TOPK = 256, 768, 100000, 5
KB = 2000
NBLK = K // KB
EPS = 1e-8
NEG_INF = float("-inf")

NFULL = KB // 128
TAILW = KB - NFULL * 128
NCH = NFULL + (1 if TAILW else 0)
RUNCH = NCH


def _topk_body(content_ref, keys_ref, spk_ref, tgt_ref, idx_out, w_out,
               run_vals, run_idx, qn_s):
    pid = pl.program_id(0)

    @pl.when(pid == 0)
    def _init():
        run_vals[...] = jnp.full((128, B), NEG_INF, jnp.float32)
        run_idx[...] = jnp.zeros((128, B), jnp.int32)
        content = content_ref[...]
        rn_q = 1.0 / (jnp.sqrt(jnp.sum(content * content, axis=1,
                                       keepdims=True)) + EPS)
        qn_s[...] = content * rn_q

    qn = qn_s[...]
    keys = keys_ref[...]
    ones_row = jnp.full((1, D), 1.0, jnp.float32)
    ssq = lax.dot_general(keys * keys, ones_row, (((1,), (1,)), ((), ())),
                          preferred_element_type=jnp.float32)
    rk = 1.0 / (jnp.sqrt(ssq) + EPS)

    sims = lax.dot_general(keys, qn, (((1,), (1,)), ((), ())),
                           preferred_element_type=jnp.float32)

    spk = spk_ref[...]
    tgt = tgt_ref[...]
    masked = jnp.where(spk == tgt, sims * rk, NEG_INF)

    run_old = run_vals[...]
    ri_old = run_idx[...]
    sub = lax.broadcasted_iota(jnp.int32, (128, B), 0)

    def chunk(j):
        if j == RUNCH:
            return run_old
        c = masked[j * 128:min((j + 1) * 128, KB), :]
        if c.shape[0] < 128:
            c = jnp.concatenate(
                [c, jnp.full((128 - c.shape[0], B), NEG_INF, jnp.float32)],
                axis=0)
        return c

    M = run_old
    G = jnp.full((128, B), RUNCH, jnp.int32)
    for j in range(NCH):
        c = chunk(j)
        upd = c > M
        M = jnp.where(upd, c, M)
        G = jnp.where(upd, j, G)

    nv = jnp.full((128, B), NEG_INF, jnp.float32)
    ni = jnp.zeros((128, B), jnp.int32)
    for t in range(TOPK):
        v = jnp.max(M, axis=0, keepdims=True)
        eq = M == v
        l = jnp.min(jnp.where(eq, sub, 128), axis=0, keepdims=True)
        onehot = sub == l
        g = jnp.sum(jnp.where(onehot, G, 0), axis=0, keepdims=True)
        ri_sel = jnp.sum(jnp.where(onehot, ri_old, 0), axis=0, keepdims=True)
        gidx = jnp.where(g < RUNCH, pid * KB + g * 128 + l, ri_sel)
        nv = jnp.where(sub == t, v, nv)
        ni = jnp.where(sub == t, gidx, ni)
        best = jnp.full((1, B), NEG_INF, jnp.float32)
        bestj = jnp.full((1, B), RUNCH, jnp.int32)
        for j in range(NCH + 1):
            colv = jnp.max(jnp.where(onehot, chunk(j), NEG_INF),
                           axis=0, keepdims=True)
            colv = jnp.where(colv < v, colv, NEG_INF)
            upd = colv > best
            best = jnp.where(upd, colv, best)
            bestj = jnp.where(upd, j, bestj)
        M = jnp.where(onehot, best, M)
        G = jnp.where(onehot, bestj, G)

    run_vals[...] = nv
    run_idx[...] = ni

    @pl.when(pid == NBLK - 1)
    def _fini():
        finite = (nv > NEG_INF) & (sub < TOPK)
        valid = jnp.where(finite, 1.0, 0.0)
        denom = jnp.maximum(jnp.sum(valid, axis=0, keepdims=True), 1.0)
        w = valid / denom
        idx_out[...] = ni[:8, :]
        w_out[...] = w[:8, :]


def _masked_topk(content, keys, spk, tgt):
    spk2 = spk.reshape(K, 1)
    tgt2 = tgt.reshape(1, B)
    return pl.pallas_call(
        _topk_body,
        grid=(NBLK,),
        in_specs=[
            pl.BlockSpec((B, D), lambda i: (0, 0)),
            pl.BlockSpec((KB, D), lambda i: (i, 0)),
            pl.BlockSpec((KB, 1), lambda i: (i, 0)),
            pl.BlockSpec((1, B), lambda i: (0, 0)),
        ],
        out_specs=[
            pl.BlockSpec((8, B), lambda i: (0, 0)),
            pl.BlockSpec((8, B), lambda i: (0, 0)),
        ],
        out_shape=[
            jax.ShapeDtypeStruct((8, B), jnp.int32),
            jax.ShapeDtypeStruct((8, B), jnp.float32),
        ],
        scratch_shapes=[
            pltpu.VMEM((128, B), jnp.float32),
            pltpu.VMEM((128, B), jnp.int32),
            pltpu.VMEM((B, D), jnp.float32),
        ],
        compiler_params=pltpu.CompilerParams(
            dimension_semantics=("arbitrary",)),
    )(content, keys, spk2, tgt2)


def _sc_gather(table, idx_flat):
    info = plsc.get_sparse_core_info()
    nc, ns = info.num_cores, info.num_subcores
    nw = nc * ns
    n = idx_flat.shape[0]
    b_per_w = n // nw
    mesh = plsc.VectorSubcoreMesh(core_axis_name="c", subcore_axis_name="s")

    @functools.partial(
        pl.kernel,
        mesh=mesh,
        out_type=jax.ShapeDtypeStruct((n, D), jnp.float32),
        scratch_types=[
            pltpu.VMEM((b_per_w,), jnp.int32),
            pltpu.VMEM((b_per_w, D), jnp.float32),
            pltpu.SemaphoreType.DMA,
        ],
    )
    def gather_kernel(table_hbm, idx_hbm, out_hbm, idx_v, rows_v, sem):
        wid = lax.axis_index("s") * nc + lax.axis_index("c")
        base = wid * b_per_w
        pltpu.sync_copy(idx_hbm.at[pl.ds(base, b_per_w)], idx_v)
        pltpu.async_copy(table_hbm.at[idx_v], rows_v, sem).wait()
        pltpu.sync_copy(rows_v, out_hbm.at[pl.ds(base, b_per_w)])

    return gather_kernel(table, idx_flat)


def _mlp_body(content_ref, g_ref, w_ref, W1_ref, b1_ref, W2_ref, b2_ref,
              out_ref):
    content = content_ref[...]
    w8 = w_ref[...]
    rmean = jnp.zeros((B, D), jnp.float32)
    sumw = jnp.zeros((B, 1), jnp.float32)
    for t in range(TOPK):
        wt = w8[t].reshape(B, 1)
        rmean = rmean + g_ref[t] * wt
        sumw = sumw + wt
    has_any = sumw > 0.5

    W1 = W1_ref[...]
    h = lax.dot_general(content, W1[:, :D], (((1,), (1,)), ((), ())),
                        preferred_element_type=jnp.float32)
    h = h + lax.dot_general(rmean, W1[:, D:], (((1,), (1,)), ((), ())),
                            preferred_element_type=jnp.float32)
    h = jnp.maximum(h + b1_ref[...], 0.0)
    out = lax.dot_general(h, W2_ref[...], (((1,), (1,)), ((), ())),
                          preferred_element_type=jnp.float32)
    out = out + b2_ref[...]
    out_ref[...] = jnp.where(has_any, out, content)


def _mlp(content, gathered, w8, W1, b1, W2, b2):
    return pl.pallas_call(
        _mlp_body,
        out_shape=jax.ShapeDtypeStruct((B, D), jnp.float32),
    )(content, gathered, w8, W1, b1.reshape(1, D), W2, b2.reshape(1, D))


@jax.jit
def kernel(content_features, training_features, W1, b1, W2, b2,
           target_speaker_id, speaker_ids):
    top_idx8, top_w8 = _masked_topk(content_features, training_features,
                                    speaker_ids.astype(jnp.int32),
                                    target_speaker_id.astype(jnp.int32))
    idx_flat = top_idx8[:TOPK].reshape(B * TOPK)
    gathered = _sc_gather(training_features, idx_flat)
    g3 = gathered.reshape(TOPK, B, D)
    return _mlp(content_features, g3, top_w8, W1, b1, W2, b2)


# speedup vs baseline: 2.8225x; 1.0313x over previous
import functools

import jax
import jax.numpy as jnp
from jax import lax
from jax.experimental import pallas as pl
from jax.experimental.pallas import tpu as pltpu
from jax.experimental.pallas import tpu_sc as plsc

B, D, K, TOPK = 256, 768, 100000, 5
KB = 2000
NBLK = K // KB
EPS = 1e-8
NEG_INF = float("-inf")

NFULL = KB // 128
TAILW = KB - NFULL * 128
NCH = NFULL + (1 if TAILW else 0)
RUNCH = NCH


def _topk_body(content_ref, keys_ref, spk_ref, tgt_ref, idx_out, w_out,
               run_vals, run_idx, qn_s):
    pid = pl.program_id(0)

    @pl.when(pid == 0)
    def _init():
        run_vals[...] = jnp.full((128, B), NEG_INF, jnp.float32)
        run_idx[...] = jnp.zeros((128, B), jnp.int32)
        content = content_ref[...]
        qn_s[...] = content / (jnp.sqrt(jnp.sum(content * content, axis=1,
                                                keepdims=True)) + EPS)

    qn = qn_s[...]
    keys = keys_ref[...]
    ssq = jnp.sum(keys * keys, axis=1, keepdims=True)
    kn = keys / (jnp.sqrt(ssq) + EPS)

    sims = lax.dot_general(kn, qn, (((1,), (1,)), ((), ())),
                           preferred_element_type=jnp.float32)

    spk = spk_ref[...]
    tgt = tgt_ref[...]
    masked = jnp.where(spk == tgt, sims, NEG_INF)

    run_old = run_vals[...]
    ri_old = run_idx[...]
    sub = lax.broadcasted_iota(jnp.int32, (128, B), 0)

    def chunk(j):
        if j == RUNCH:
            return run_old
        c = masked[j * 128:min((j + 1) * 128, KB), :]
        if c.shape[0] < 128:
            c = jnp.concatenate(
                [c, jnp.full((128 - c.shape[0], B), NEG_INF, jnp.float32)],
                axis=0)
        return c

    M = run_old
    G = jnp.full((128, B), RUNCH, jnp.int32)
    for j in range(NCH):
        c = chunk(j)
        upd = c > M
        M = jnp.where(upd, c, M)
        G = jnp.where(upd, j, G)

    nv = jnp.full((128, B), NEG_INF, jnp.float32)
    ni = jnp.zeros((128, B), jnp.int32)
    for t in range(TOPK):
        v = jnp.max(M, axis=0, keepdims=True)
        eq = M == v
        l = jnp.min(jnp.where(eq, sub, 128), axis=0, keepdims=True)
        onehot = sub == l
        g = jnp.sum(jnp.where(onehot, G, 0), axis=0, keepdims=True)
        ri_sel = jnp.sum(jnp.where(onehot, ri_old, 0), axis=0, keepdims=True)
        gidx = jnp.where(g < RUNCH, pid * KB + g * 128 + l, ri_sel)
        nv = jnp.where(sub == t, v, nv)
        ni = jnp.where(sub == t, gidx, ni)
        best = jnp.full((1, B), NEG_INF, jnp.float32)
        bestj = jnp.full((1, B), RUNCH, jnp.int32)
        for j in range(NCH + 1):
            colv = jnp.max(jnp.where(onehot, chunk(j), NEG_INF),
                           axis=0, keepdims=True)
            colv = jnp.where(colv < v, colv, NEG_INF)
            upd = colv > best
            best = jnp.where(upd, colv, best)
            bestj = jnp.where(upd, j, bestj)
        M = jnp.where(onehot, best, M)
        G = jnp.where(onehot, bestj, G)

    run_vals[...] = nv
    run_idx[...] = ni

    @pl.when(pid == NBLK - 1)
    def _fini():
        finite = (nv > NEG_INF) & (sub < TOPK)
        valid = jnp.where(finite, 1.0, 0.0)
        denom = jnp.maximum(jnp.sum(valid, axis=0, keepdims=True), 1.0)
        w = valid / denom
        idx_out[...] = ni[:8, :]
        w_out[...] = w[:8, :]


def _masked_topk(content, keys, spk, tgt):
    spk2 = spk.reshape(K, 1)
    tgt2 = tgt.reshape(1, B)
    return pl.pallas_call(
        _topk_body,
        grid=(NBLK,),
        in_specs=[
            pl.BlockSpec((B, D), lambda i: (0, 0)),
            pl.BlockSpec((KB, D), lambda i: (i, 0)),
            pl.BlockSpec((KB, 1), lambda i: (i, 0)),
            pl.BlockSpec((1, B), lambda i: (0, 0)),
        ],
        out_specs=[
            pl.BlockSpec((8, B), lambda i: (0, 0)),
            pl.BlockSpec((8, B), lambda i: (0, 0)),
        ],
        out_shape=[
            jax.ShapeDtypeStruct((8, B), jnp.int32),
            jax.ShapeDtypeStruct((8, B), jnp.float32),
        ],
        scratch_shapes=[
            pltpu.VMEM((128, B), jnp.float32),
            pltpu.VMEM((128, B), jnp.int32),
            pltpu.VMEM((B, D), jnp.float32),
        ],
        compiler_params=pltpu.CompilerParams(
            dimension_semantics=("arbitrary",)),
    )(content, keys, spk2, tgt2)


def _sc_gather(table, idx_flat):
    info = plsc.get_sparse_core_info()
    nc, ns = info.num_cores, info.num_subcores
    nw = nc * ns
    n = idx_flat.shape[0]
    b_per_w = n // nw
    mesh = plsc.VectorSubcoreMesh(core_axis_name="c", subcore_axis_name="s")

    @functools.partial(
        pl.kernel,
        mesh=mesh,
        out_type=jax.ShapeDtypeStruct((n, D), jnp.float32),
        scratch_types=[
            pltpu.VMEM((b_per_w,), jnp.int32),
            pltpu.VMEM((b_per_w, D), jnp.float32),
            pltpu.SemaphoreType.DMA,
        ],
    )
    def gather_kernel(table_hbm, idx_hbm, out_hbm, idx_v, rows_v, sem):
        wid = lax.axis_index("s") * nc + lax.axis_index("c")
        base = wid * b_per_w
        pltpu.sync_copy(idx_hbm.at[pl.ds(base, b_per_w)], idx_v)
        pltpu.async_copy(table_hbm.at[idx_v], rows_v, sem).wait()
        pltpu.sync_copy(rows_v, out_hbm.at[pl.ds(base, b_per_w)])

    return gather_kernel(table, idx_flat)


def _mlp_body(content_ref, g_ref, w_ref, W1_ref, b1_ref, W2_ref, b2_ref,
              out_ref):
    content = content_ref[...]
    w8 = w_ref[...]
    rmean = jnp.zeros((B, D), jnp.float32)
    sumw = jnp.zeros((B, 1), jnp.float32)
    for t in range(TOPK):
        wt = w8[t].reshape(B, 1)
        rmean = rmean + g_ref[t] * wt
        sumw = sumw + wt
    has_any = sumw > 0.5

    W1 = W1_ref[...]
    h = lax.dot_general(content, W1[:, :D], (((1,), (1,)), ((), ())),
                        preferred_element_type=jnp.float32)
    h = h + lax.dot_general(rmean, W1[:, D:], (((1,), (1,)), ((), ())),
                            preferred_element_type=jnp.float32)
    h = jnp.maximum(h + b1_ref[...], 0.0)
    out = lax.dot_general(h, W2_ref[...], (((1,), (1,)), ((), ())),
                          preferred_element_type=jnp.float32)
    out = out + b2_ref[...]
    out_ref[...] = jnp.where(has_any, out, content)


def _mlp(content, gathered, w8, W1, b1, W2, b2):
    return pl.pallas_call(
        _mlp_body,
        out_shape=jax.ShapeDtypeStruct((B, D), jnp.float32),
    )(content, gathered, w8, W1, b1.reshape(1, D), W2, b2.reshape(1, D))


@jax.jit
def kernel(content_features, training_features, W1, b1, W2, b2,
           target_speaker_id, speaker_ids):
    top_idx8, top_w8 = _masked_topk(content_features, training_features,
                                    speaker_ids.astype(jnp.int32),
                                    target_speaker_id.astype(jnp.int32))
    idx_flat = top_idx8[:TOPK].reshape(B * TOPK)
    gathered = _sc_gather(training_features, idx_flat)
    g3 = gathered.reshape(TOPK, B, D)
    return _mlp(content_features, g3, top_w8, W1, b1, W2, b2)


# speedup vs baseline: 3.0040x; 1.0643x over previous
import functools

import jax
import jax.numpy as jnp
from jax import lax
from jax.experimental import pallas as pl
from jax.experimental.pallas import tpu as pltpu
from jax.experimental.pallas import tpu_sc as plsc

B, D, K, TOPK = 256, 768, 100000, 5
KB = 4000
NBLK = K // KB
EPS = 1e-8
NEG_INF = float("-inf")

NFULL = KB // 128
TAILW = KB - NFULL * 128
NCH = NFULL + (1 if TAILW else 0)
RUNCH = NCH


def _topk_body(content_ref, keys_ref, spk_ref, tgt_ref, idx_out, w_out,
               run_vals, run_idx, qn_s):
    pid = pl.program_id(0)

    @pl.when(pid == 0)
    def _init():
        run_vals[...] = jnp.full((128, B), NEG_INF, jnp.float32)
        run_idx[...] = jnp.zeros((128, B), jnp.int32)
        content = content_ref[...]
        qn_s[...] = content / (jnp.sqrt(jnp.sum(content * content, axis=1,
                                                keepdims=True)) + EPS)

    qn = qn_s[...]
    keys = keys_ref[...]
    ssq = jnp.sum(keys * keys, axis=1, keepdims=True)
    kn = keys / (jnp.sqrt(ssq) + EPS)

    sims = lax.dot_general(kn, qn, (((1,), (1,)), ((), ())),
                           preferred_element_type=jnp.float32)

    spk = spk_ref[...]
    tgt = tgt_ref[...]
    masked = jnp.where(spk == tgt, sims, NEG_INF)

    run_old = run_vals[...]
    ri_old = run_idx[...]
    sub = lax.broadcasted_iota(jnp.int32, (128, B), 0)

    def chunk(j):
        if j == RUNCH:
            return run_old
        c = masked[j * 128:min((j + 1) * 128, KB), :]
        if c.shape[0] < 128:
            c = jnp.concatenate(
                [c, jnp.full((128 - c.shape[0], B), NEG_INF, jnp.float32)],
                axis=0)
        return c

    M = run_old
    G = jnp.full((128, B), RUNCH, jnp.int32)
    for j in range(NCH):
        c = chunk(j)
        upd = c > M
        M = jnp.where(upd, c, M)
        G = jnp.where(upd, j, G)

    nv = jnp.full((128, B), NEG_INF, jnp.float32)
    ni = jnp.zeros((128, B), jnp.int32)
    for t in range(TOPK):
        v = jnp.max(M, axis=0, keepdims=True)
        eq = M == v
        l = jnp.min(jnp.where(eq, sub, 128), axis=0, keepdims=True)
        onehot = sub == l
        g = jnp.sum(jnp.where(onehot, G, 0), axis=0, keepdims=True)
        ri_sel = jnp.sum(jnp.where(onehot, ri_old, 0), axis=0, keepdims=True)
        gidx = jnp.where(g < RUNCH, pid * KB + g * 128 + l, ri_sel)
        nv = jnp.where(sub == t, v, nv)
        ni = jnp.where(sub == t, gidx, ni)
        best = jnp.full((1, B), NEG_INF, jnp.float32)
        bestj = jnp.full((1, B), RUNCH, jnp.int32)
        for j in range(NCH + 1):
            colv = jnp.max(jnp.where(onehot, chunk(j), NEG_INF),
                           axis=0, keepdims=True)
            colv = jnp.where(colv < v, colv, NEG_INF)
            upd = colv > best
            best = jnp.where(upd, colv, best)
            bestj = jnp.where(upd, j, bestj)
        M = jnp.where(onehot, best, M)
        G = jnp.where(onehot, bestj, G)

    run_vals[...] = nv
    run_idx[...] = ni

    @pl.when(pid == NBLK - 1)
    def _fini():
        finite = (nv > NEG_INF) & (sub < TOPK)
        valid = jnp.where(finite, 1.0, 0.0)
        denom = jnp.maximum(jnp.sum(valid, axis=0, keepdims=True), 1.0)
        w = valid / denom
        idx_out[...] = ni[:8, :]
        w_out[...] = w[:8, :]


def _masked_topk(content, keys, spk, tgt):
    spk2 = spk.reshape(K, 1)
    tgt2 = tgt.reshape(1, B)
    return pl.pallas_call(
        _topk_body,
        grid=(NBLK,),
        in_specs=[
            pl.BlockSpec((B, D), lambda i: (0, 0)),
            pl.BlockSpec((KB, D), lambda i: (i, 0)),
            pl.BlockSpec((KB, 1), lambda i: (i, 0)),
            pl.BlockSpec((1, B), lambda i: (0, 0)),
        ],
        out_specs=[
            pl.BlockSpec((8, B), lambda i: (0, 0)),
            pl.BlockSpec((8, B), lambda i: (0, 0)),
        ],
        out_shape=[
            jax.ShapeDtypeStruct((8, B), jnp.int32),
            jax.ShapeDtypeStruct((8, B), jnp.float32),
        ],
        scratch_shapes=[
            pltpu.VMEM((128, B), jnp.float32),
            pltpu.VMEM((128, B), jnp.int32),
            pltpu.VMEM((B, D), jnp.float32),
        ],
        compiler_params=pltpu.CompilerParams(
            dimension_semantics=("arbitrary",)),
    )(content, keys, spk2, tgt2)


def _sc_gather(table, idx_flat):
    info = plsc.get_sparse_core_info()
    nc, ns = info.num_cores, info.num_subcores
    nw = nc * ns
    n = idx_flat.shape[0]
    b_per_w = n // nw
    mesh = plsc.VectorSubcoreMesh(core_axis_name="c", subcore_axis_name="s")

    @functools.partial(
        pl.kernel,
        mesh=mesh,
        out_type=jax.ShapeDtypeStruct((n, D), jnp.float32),
        scratch_types=[
            pltpu.VMEM((b_per_w,), jnp.int32),
            pltpu.VMEM((b_per_w, D), jnp.float32),
            pltpu.SemaphoreType.DMA,
        ],
    )
    def gather_kernel(table_hbm, idx_hbm, out_hbm, idx_v, rows_v, sem):
        wid = lax.axis_index("s") * nc + lax.axis_index("c")
        base = wid * b_per_w
        pltpu.sync_copy(idx_hbm.at[pl.ds(base, b_per_w)], idx_v)
        pltpu.async_copy(table_hbm.at[idx_v], rows_v, sem).wait()
        pltpu.sync_copy(rows_v, out_hbm.at[pl.ds(base, b_per_w)])

    return gather_kernel(table, idx_flat)


def _mlp_body(content_ref, g_ref, w_ref, W1_ref, b1_ref, W2_ref, b2_ref,
              out_ref):
    content = content_ref[...]
    w8 = w_ref[...]
    rmean = jnp.zeros((B, D), jnp.float32)
    sumw = jnp.zeros((B, 1), jnp.float32)
    for t in range(TOPK):
        wt = w8[t].reshape(B, 1)
        rmean = rmean + g_ref[t] * wt
        sumw = sumw + wt
    has_any = sumw > 0.5

    W1 = W1_ref[...]
    h = lax.dot_general(content, W1[:, :D], (((1,), (1,)), ((), ())),
                        preferred_element_type=jnp.float32)
    h = h + lax.dot_general(rmean, W1[:, D:], (((1,), (1,)), ((), ())),
                            preferred_element_type=jnp.float32)
    h = jnp.maximum(h + b1_ref[...], 0.0)
    out = lax.dot_general(h, W2_ref[...], (((1,), (1,)), ((), ())),
                          preferred_element_type=jnp.float32)
    out = out + b2_ref[...]
    out_ref[...] = jnp.where(has_any, out, content)


def _mlp(content, gathered, w8, W1, b1, W2, b2):
    return pl.pallas_call(
        _mlp_body,
        out_shape=jax.ShapeDtypeStruct((B, D), jnp.float32),
    )(content, gathered, w8, W1, b1.reshape(1, D), W2, b2.reshape(1, D))


@jax.jit
def kernel(content_features, training_features, W1, b1, W2, b2,
           target_speaker_id, speaker_ids):
    top_idx8, top_w8 = _masked_topk(content_features, training_features,
                                    speaker_ids.astype(jnp.int32),
                                    target_speaker_id.astype(jnp.int32))
    idx_flat = top_idx8[:TOPK].reshape(B * TOPK)
    gathered = _sc_gather(training_features, idx_flat)
    g3 = gathered.reshape(TOPK, B, D)
    return _mlp(content_features, g3, top_w8, W1, b1, W2, b2)
